# Initial kernel scaffold; baseline (speedup 1.0000x reference)
import jax
import jax.numpy as jnp
from jax.experimental import pallas as pl


def kernel(x, mask_idx, mask_token, W_cont, b_cont):
    raise NotImplementedError("write your pallas kernel here")


# speedup vs baseline: 6.2653x; 6.2653x over previous
import functools

import jax
import jax.numpy as jnp
from jax import lax
from jax.experimental import pallas as pl
from jax.experimental.pallas import tpu as pltpu
from jax.experimental.pallas import tpu_sc as plsc

N_ROWS = 100000
D = 256
M = 15000

NC = 2
NS = 16
NW = NC * NS
LANES = 16
NJ = D // LANES

B_PER_W = 480
M_PAD = NW * B_PER_W
CHUNK = 120
NCHUNK = B_PER_W // CHUNK

ROWS_BLK = 3000


def _pred_row_body(t_ref, w_ref, b_ref, o_ref):
    o_ref[...] = (
        jnp.dot(t_ref[...], w_ref[...], preferred_element_type=jnp.float32)
        + b_ref[...]
    )


def _bcast_body(p_ref, o_ref):
    o_ref[...] = jnp.broadcast_to(p_ref[...], o_ref.shape)


_sc_mesh = plsc.VectorSubcoreMesh(core_axis_name="c", subcore_axis_name="s")


@functools.partial(
    pl.kernel,
    mesh=_sc_mesh,
    out_type=jax.ShapeDtypeStruct((NW, LANES), jnp.float32),
    scratch_types=[
        pltpu.VMEM((NCHUNK, CHUNK), jnp.int32),
        pltpu.VMEM((B_PER_W, D), jnp.float32),
        pltpu.VMEM((D,), jnp.float32),
        pltpu.VMEM((LANES,), jnp.float32),
        pltpu.SemaphoreType.DMA,
    ],
)
def _sc_mse_partials(x_hbm, idx_hbm, p_hbm, out_hbm, idx_v, rows_v, p_v, part_v, sem):
    wid = lax.axis_index("s") * NC + lax.axis_index("c")
    pltpu.sync_copy(idx_hbm.at[wid], idx_v)
    pltpu.sync_copy(p_hbm, p_v)

    copies = []
    for c in range(NCHUNK):
        copies.append(
            pltpu.async_copy(
                x_hbm.at[idx_v.at[c]],
                rows_v.at[pl.ds(c * CHUNK, CHUNK)],
                sem,
            )
        )
    for cp in copies:
        cp.wait()

    base = wid * B_PER_W
    n_valid = jnp.clip(M - base, 0, B_PER_W)

    pj = [p_v[pl.ds(j * LANES, LANES)] for j in range(NJ)]

    def body(r, accs):
        new = []
        for j in range(NJ):
            d = rows_v[r, pl.ds(j * LANES, LANES)] - pj[j]
            new.append(accs[j] + d * d)
        return tuple(new)

    accs = lax.fori_loop(
        0, n_valid, body,
        tuple(jnp.zeros((LANES,), jnp.float32) for _ in range(NJ)),
    )

    tot = accs[0]
    for j in range(1, NJ):
        tot = tot + accs[j]
    part_v[...] = tot
    pltpu.sync_copy(part_v, out_hbm.at[wid])


def kernel(x, mask_idx, mask_token, W_cont, b_cont):
    p_row = pl.pallas_call(
        _pred_row_body,
        out_shape=jax.ShapeDtypeStruct((1, D), jnp.float32),
    )(mask_token.reshape(1, D), W_cont, b_cont.reshape(1, D))

    pred_cont = pl.pallas_call(
        _bcast_body,
        grid=(M // ROWS_BLK,),
        in_specs=[pl.BlockSpec((1, D), lambda i: (0, 0))],
        out_specs=pl.BlockSpec((ROWS_BLK, D), lambda i: (i, 0)),
        out_shape=jax.ShapeDtypeStruct((M, D), jnp.float32),
    )(p_row)

    idx_pad = jnp.concatenate(
        [mask_idx, jnp.zeros((M_PAD - M,), jnp.int32)]
    ).reshape(NW, NCHUNK, CHUNK)
    partials = _sc_mse_partials(x, idx_pad, p_row.reshape(D))

    total_loss = jnp.sum(partials) / (M * D)
    return (total_loss, pred_cont)
